# Initial kernel scaffold; baseline (speedup 1.0000x reference)
import jax
import jax.numpy as jnp
from jax.experimental import pallas as pl


def kernel(token_ids, weight):
    raise NotImplementedError("write your pallas kernel here")


# speedup vs baseline: 2.9812x; 2.9812x over previous
import functools

import jax
import jax.numpy as jnp
from jax import lax
from jax.experimental import pallas as pl
from jax.experimental.pallas import tpu as pltpu
from jax.experimental.pallas import tpu_sc as plsc

VOCAB = 100000
D = 128
NC, NS = 2, 16
NW = NC * NS
L = 128


def _emb_kernel(n_steps: int):
    mesh = plsc.VectorSubcoreMesh(
        core_axis_name="c", subcore_axis_name="s", num_cores=NC, num_subcores=NS
    )
    b_total = NW * n_steps * L

    @functools.partial(
        pl.kernel,
        out_type=jax.ShapeDtypeStruct((b_total, D), jnp.float32),
        mesh=mesh,
        scratch_types=[
            pltpu.VMEM((n_steps, L), jnp.int32),
            pltpu.VMEM((L, D), jnp.float32),
            pltpu.SemaphoreType.DMA,
        ],
    )
    def body(tbl_hbm, idx_hbm, out_hbm, idx_v, rows_v, sem):
        wid = lax.axis_index("s") * NC + lax.axis_index("c")
        base = wid * (n_steps * L)
        pltpu.sync_copy(idx_hbm.at[wid], idx_v)

        def step(j, carry):
            pltpu.async_copy(tbl_hbm.at[idx_v.at[j]], rows_v, sem).wait()
            pltpu.sync_copy(rows_v, out_hbm.at[pl.ds(base + j * L, L)])
            return carry

        lax.fori_loop(0, n_steps, step, 0)

    return body


def kernel(token_ids, weight):
    B, S = token_ids.shape
    total = B * S
    assert total % (NW * L) == 0
    n_steps = total // (NW * L)
    idx = token_ids.astype(jnp.int32).reshape(NW, n_steps, L)
    out = _emb_kernel(n_steps)(weight, idx)
    return out.reshape(B, S, D)


# speedup vs baseline: 3.3325x; 1.1179x over previous
import functools

import jax
import jax.numpy as jnp
from jax import lax
from jax.experimental import pallas as pl
from jax.experimental.pallas import tpu as pltpu
from jax.experimental.pallas import tpu_sc as plsc

VOCAB = 100000
D = 128
NC, NS = 2, 16
NW = NC * NS
L = 128
NBUF = 5


def _emb_kernel(n_steps: int):
    assert n_steps % NBUF == 0
    n_outer = n_steps // NBUF
    mesh = plsc.VectorSubcoreMesh(
        core_axis_name="c", subcore_axis_name="s", num_cores=NC, num_subcores=NS
    )
    b_total = NW * n_steps * L

    @functools.partial(
        pl.kernel,
        out_type=jax.ShapeDtypeStruct((b_total, D), jnp.float32),
        mesh=mesh,
        scratch_types=[
            pltpu.VMEM((n_steps, L), jnp.int32),
            pltpu.VMEM((NBUF, L, D), jnp.float32),
            pltpu.SemaphoreType.DMA((NBUF,)),
            pltpu.SemaphoreType.DMA((NBUF,)),
        ],
    )
    def body(tbl_hbm, idx_hbm, out_hbm, idx_v, rows_v, gsem, wsem):
        wid = lax.axis_index("s") * NC + lax.axis_index("c")
        base = wid * (n_steps * L)
        pltpu.sync_copy(idx_hbm.at[wid], idx_v)

        def gather(j, b):
            return pltpu.make_async_copy(
                tbl_hbm.at[idx_v.at[j]], rows_v.at[b], gsem.at[b]
            )

        def write(j, b):
            return pltpu.make_async_copy(
                rows_v.at[b], out_hbm.at[pl.ds(base + j * L, L)], wsem.at[b]
            )

        for b in range(NBUF):
            gather(b, b).start()

        def outer(jo, carry):
            j0 = jo * NBUF
            for b in range(NBUF):
                gather(j0 + b, b).wait()
                write(j0 + b, b).start()
            for b in range(NBUF):
                @pl.when(j0 + b + NBUF < n_steps)
                def _():
                    write(j0 + b, b).wait()
                    gather(j0 + b + NBUF, b).start()
            return carry

        lax.fori_loop(0, n_outer, outer, 0)

        for b in range(NBUF):
            write((n_outer - 1) * NBUF + b, b).wait()

    return body


def kernel(token_ids, weight):
    B, S = token_ids.shape
    total = B * S
    assert total % (NW * L) == 0
    n_steps = total // (NW * L)
    idx = token_ids.astype(jnp.int32).reshape(NW, n_steps, L)
    out = _emb_kernel(n_steps)(weight, idx)
    return out.reshape(B, S, D)
